# baseline (speedup 1.0000x reference)
import functools

import jax
import jax.numpy as jnp
from jax import lax
from jax.experimental import pallas as pl
from jax.experimental.pallas import tpu as pltpu
from jax.experimental.pallas import tpu_sc as plsc

NUM_ROWS = 1000
DIM = 64
BATCH = 4096
SEQ = 200

NC = 2
NS = 16
NW = NC * NS
BPW = BATCH // NW
IDX_PER_W = BPW * SEQ
L = 16
QW = DIM // L
NXC = 4
XC = BPW // NXC
BPAD = BPW + 1


@functools.partial(
    pl.kernel,
    out_type=jax.ShapeDtypeStruct((SEQ * 8, NW, 8, 128), jnp.float32),
    mesh=plsc.VectorSubcoreMesh(core_axis_name="c", subcore_axis_name="s"),
    scratch_types=[
        pltpu.VMEM_SHARED((NUM_ROWS, DIM), jnp.float32),
        pltpu.VMEM((XC * SEQ,), jnp.int32),
        pltpu.VMEM((SEQ, BPW), jnp.int32),
        pltpu.VMEM((BPW, DIM), jnp.float32),
        pltpu.VMEM((BPW, DIM), jnp.float32),
        pltpu.VMEM((8, 8, BPAD), jnp.float32),
        pltpu.VMEM((8, 8, BPAD), jnp.float32),
        pltpu.SemaphoreType.DMA,
        pltpu.SemaphoreType.DMA,
        pltpu.SemaphoreType.DMA,
        pltpu.SemaphoreType.DMA,
    ],
    compiler_params=pltpu.CompilerParams(use_tc_tiling_on_sc=False,
                                         needs_layout_passes=False),
)
def _lookup(x_hbm, table_hbm, out_hbm, table_sp, xc_v, xt_v,
            a0, a1, b0, b1, g0, g1, o0, o1):
    wid = lax.axis_index("s") * NC + lax.axis_index("c")

    @pl.when(lax.axis_index("s") == 0)
    def _():
        pltpu.sync_copy(table_hbm, table_sp)

    plsc.subcore_barrier()

    lane = lax.iota(jnp.int32, L)
    dcol = [lane + q * L for q in range(QW)]
    dtv = [(lane + q * L) // 8 for q in range(QW)]
    subv = [(lane + q * L) % 8 for q in range(QW)]
    lane_seq = lane * SEQ

    for c in range(NXC):
        pltpu.sync_copy(
            x_hbm.at[pl.ds(wid * IDX_PER_W + c * (XC * SEQ), XC * SEQ)],
            xc_v)

        @plsc.parallel_loop(0, SEQ)
        def _(j):
            for g in range(XC // L):
                idx = lane_seq + (g * L * SEQ) + j
                v = plsc.load_gather(xc_v, [idx])
                xt_v[j, pl.ds(c * XC + g * L, L)] = v

    def fire_gather(j, a, sem):
        src = table_sp.at[xt_v.at[j, pl.ds(0, BPW)]]
        pltpu.make_async_copy(src, a, sem).start()

    def wait_gather(a, sem):
        src = table_sp.at[xt_v.at[0, pl.ds(0, BPW)]]
        pltpu.make_async_copy(src, a, sem).wait()

    def transpose(a, b):
        @plsc.parallel_loop(0, BPW, step=16)
        def _(i0):
            for u in range(16):
                i = i0 + u
                icol = jnp.full((L,), i, jnp.int32)
                for q in range(QW):
                    v = a[i, pl.ds(q * L, L)]
                    plsc.store_scatter(b, [dtv[q], subv[q], icol], v)

    def start_out(j, b, sem):
        pltpu.make_async_copy(b.at[:, :, pl.ds(0, BPW)],
                              out_hbm.at[pl.ds(j * 8, 8), wid],
                              sem).start()

    def wait_out(b, sem):
        pltpu.make_async_copy(b.at[:, :, pl.ds(0, BPW)],
                              out_hbm.at[pl.ds(0, 8), wid],
                              sem).wait()

    fire_gather(0, a0, g0)
    fire_gather(1, a1, g1)

    @pl.loop(0, SEQ, step=2)
    def _(j):
        wait_gather(a0, g0)

        @pl.when(j >= 2)
        def _():
            wait_out(b0, o0)

        transpose(a0, b0)

        @pl.when(j + 2 < SEQ)
        def _():
            fire_gather(j + 2, a0, g0)

        start_out(j, b0, o0)

        wait_gather(a1, g1)

        @pl.when(j >= 2)
        def _():
            wait_out(b1, o1)

        transpose(a1, b1)

        @pl.when(j + 3 < SEQ)
        def _():
            fire_gather(j + 3, a1, g1)

        start_out(j + 1, b1, o1)

    wait_out(b0, o0)
    wait_out(b1, o1)


def kernel(x, table):
    t = _lookup(x.reshape(-1), table)
    a5 = t.reshape(SEQ, 8, NW, 8, 128)
    a5t = jnp.transpose(a5, (0, 1, 3, 2, 4))
    out3 = a5t.reshape(SEQ, DIM, BATCH)
    return jnp.transpose(out3, (2, 0, 1))


# speedup vs baseline: 1.7217x; 1.7217x over previous
import functools

import jax
import jax.numpy as jnp
from jax import lax
from jax.experimental import pallas as pl
from jax.experimental.pallas import tpu as pltpu
from jax.experimental.pallas import tpu_sc as plsc

NUM_ROWS = 1000
DIM = 64
BATCH = 4096
SEQ = 200

NC = 2
NS = 16
NW = NC * NS
BPW = BATCH // NW
IDX_PER_W = BPW * SEQ
L = 16
QW = DIM // L
NXC = 4
XC = BPW // NXC
BPAD = BPW + 1


@functools.partial(
    pl.kernel,
    out_type=jax.ShapeDtypeStruct((SEQ * 8, NW, 8, 128), jnp.float32),
    mesh=plsc.VectorSubcoreMesh(core_axis_name="c", subcore_axis_name="s"),
    scratch_types=[
        pltpu.VMEM_SHARED((NUM_ROWS, DIM), jnp.float32),
        pltpu.VMEM((XC * SEQ,), jnp.int32),
        pltpu.VMEM((SEQ, BPW), jnp.int32),
        pltpu.VMEM((BPW, DIM), jnp.float32),
        pltpu.VMEM((BPW, DIM), jnp.float32),
        pltpu.VMEM((8, 8, BPAD), jnp.float32),
        pltpu.VMEM((8, 8, BPAD), jnp.float32),
        pltpu.SemaphoreType.DMA,
        pltpu.SemaphoreType.DMA,
        pltpu.SemaphoreType.DMA,
        pltpu.SemaphoreType.DMA,
    ],
    compiler_params=pltpu.CompilerParams(use_tc_tiling_on_sc=False,
                                         needs_layout_passes=False),
)
def _lookup(x_hbm, table_hbm, out_hbm, table_sp, xc_v, xt_v,
            a0, a1, b0, b1, g0, g1, o0, o1):
    wid = lax.axis_index("s") * NC + lax.axis_index("c")

    @pl.when(lax.axis_index("s") == 0)
    def _():
        pltpu.sync_copy(table_hbm, table_sp)

    plsc.subcore_barrier()

    lane = lax.iota(jnp.int32, L)
    dcol = [lane + q * L for q in range(QW)]
    dtv = [(lane + q * L) // 8 for q in range(QW)]
    subv = [(lane + q * L) % 8 for q in range(QW)]
    lane_seq = lane * SEQ

    for c in range(NXC):
        pltpu.sync_copy(
            x_hbm.at[pl.ds(wid * IDX_PER_W + c * (XC * SEQ), XC * SEQ)],
            xc_v)

        @plsc.parallel_loop(0, SEQ)
        def _(j):
            for g in range(XC // L):
                idx = lane_seq + (g * L * SEQ) + j
                v = plsc.load_gather(xc_v, [idx])
                xt_v[j, pl.ds(c * XC + g * L, L)] = v

    def fire_gather(j, a, sem):
        src = table_sp.at[xt_v.at[j, pl.ds(0, BPW)]]
        pltpu.make_async_copy(src, a, sem).start()

    def wait_gather(a, sem):
        src = table_sp.at[xt_v.at[0, pl.ds(0, BPW)]]
        pltpu.make_async_copy(src, a, sem).wait()

    def transpose(a, b):
        @plsc.parallel_loop(0, BPW, step=4)
        def _(i0):
            for u in range(4):
                i = i0 + u
                icol = jnp.full((L,), i, jnp.int32)
                for q in range(QW):
                    v = a[i, pl.ds(q * L, L)]
                    plsc.store_scatter(b, [dtv[q], subv[q], icol], v)

    def start_out(j, b, sem):
        pltpu.make_async_copy(b.at[:, :, pl.ds(0, BPW)],
                              out_hbm.at[pl.ds(j * 8, 8), wid],
                              sem).start()

    def wait_out(b, sem):
        pltpu.make_async_copy(b.at[:, :, pl.ds(0, BPW)],
                              out_hbm.at[pl.ds(0, 8), wid],
                              sem).wait()

    fire_gather(0, a0, g0)
    fire_gather(1, a1, g1)

    @pl.loop(0, SEQ, step=2)
    def _(j):
        wait_gather(a0, g0)

        @pl.when(j >= 2)
        def _():
            wait_out(b0, o0)

        transpose(a0, b0)

        @pl.when(j + 2 < SEQ)
        def _():
            fire_gather(j + 2, a0, g0)

        start_out(j, b0, o0)

        wait_gather(a1, g1)

        @pl.when(j >= 2)
        def _():
            wait_out(b1, o1)

        transpose(a1, b1)

        @pl.when(j + 3 < SEQ)
        def _():
            fire_gather(j + 3, a1, g1)

        start_out(j + 1, b1, o1)

    wait_out(b0, o0)
    wait_out(b1, o1)


def kernel(x, table):
    t = _lookup(x.reshape(-1), table)
    a5 = t.reshape(SEQ, 8, NW, 8, 128)
    a5t = jnp.transpose(a5, (0, 1, 3, 2, 4))
    out3 = a5t.reshape(SEQ, DIM, BATCH)
    return jnp.transpose(out3, (2, 0, 1))


# speedup vs baseline: 1.7268x; 1.0029x over previous
import functools

import jax
import jax.numpy as jnp
from jax import lax
from jax.experimental import pallas as pl
from jax.experimental.pallas import tpu as pltpu
from jax.experimental.pallas import tpu_sc as plsc

NUM_ROWS = 1000
DIM = 64
BATCH = 4096
SEQ = 200

NC = 2
NS = 16
NW = NC * NS
BPW = BATCH // NW
IDX_PER_W = BPW * SEQ
L = 16
QW = DIM // L
NXC = 4
XC = BPW // NXC
BPAD = BPW + 1


@functools.partial(
    pl.kernel,
    out_type=jax.ShapeDtypeStruct((SEQ * 8, NW, 8, 128), jnp.float32),
    mesh=plsc.VectorSubcoreMesh(core_axis_name="c", subcore_axis_name="s"),
    scratch_types=[
        pltpu.VMEM_SHARED((NUM_ROWS, DIM), jnp.float32),
        pltpu.VMEM((XC * SEQ,), jnp.int32),
        pltpu.VMEM((SEQ, BPW), jnp.int32),
        pltpu.VMEM((BPW, DIM), jnp.float32),
        pltpu.VMEM((BPW, DIM), jnp.float32),
        pltpu.VMEM((8, 8, BPAD), jnp.float32),
        pltpu.VMEM((8, 8, BPAD), jnp.float32),
        pltpu.SemaphoreType.DMA,
        pltpu.SemaphoreType.DMA,
        pltpu.SemaphoreType.DMA,
        pltpu.SemaphoreType.DMA,
    ],
    compiler_params=pltpu.CompilerParams(use_tc_tiling_on_sc=False,
                                         needs_layout_passes=False),
)
def _lookup(x_hbm, table_hbm, out_hbm, table_sp, xc_v, xt_v,
            a0, a1, b0, b1, g0, g1, o0, o1):
    wid = lax.axis_index("s") * NC + lax.axis_index("c")

    @pl.when(lax.axis_index("s") == 0)
    def _():
        pltpu.sync_copy(table_hbm, table_sp)

    plsc.subcore_barrier()

    lane = lax.iota(jnp.int32, L)
    dcol = [lane + q * L for q in range(QW)]
    dtv = [(lane + q * L) // 8 for q in range(QW)]
    subv = [(lane + q * L) % 8 for q in range(QW)]
    lane_seq = lane * SEQ

    for c in range(NXC):
        pltpu.sync_copy(
            x_hbm.at[pl.ds(wid * IDX_PER_W + c * (XC * SEQ), XC * SEQ)],
            xc_v)

        @plsc.parallel_loop(0, SEQ)
        def _(j):
            for g in range(XC // L):
                idx = lane_seq + (g * L * SEQ) + j
                v = plsc.load_gather(xc_v, [idx])
                xt_v[j, pl.ds(c * XC + g * L, L)] = v

    def fire_gather(j, a, sem):
        src = table_sp.at[xt_v.at[j, pl.ds(0, BPW)]]
        pltpu.make_async_copy(src, a, sem).start()

    def wait_gather(a, sem):
        src = table_sp.at[xt_v.at[0, pl.ds(0, BPW)]]
        pltpu.make_async_copy(src, a, sem).wait()

    def transpose(a, b):
        @plsc.parallel_loop(0, BPW, step=2)
        def _(i0):
            for u in range(2):
                i = i0 + u
                icol = jnp.full((L,), i, jnp.int32)
                for q in range(QW):
                    v = a[i, pl.ds(q * L, L)]
                    plsc.store_scatter(b, [dtv[q], subv[q], icol], v)

    def start_out(j, b, sem):
        pltpu.make_async_copy(b.at[:, :, pl.ds(0, BPW)],
                              out_hbm.at[pl.ds(j * 8, 8), wid],
                              sem).start()

    def wait_out(b, sem):
        pltpu.make_async_copy(b.at[:, :, pl.ds(0, BPW)],
                              out_hbm.at[pl.ds(0, 8), wid],
                              sem).wait()

    fire_gather(0, a0, g0)
    fire_gather(1, a1, g1)

    @pl.loop(0, SEQ, step=2)
    def _(j):
        wait_gather(a0, g0)

        @pl.when(j >= 2)
        def _():
            wait_out(b0, o0)

        transpose(a0, b0)

        @pl.when(j + 2 < SEQ)
        def _():
            fire_gather(j + 2, a0, g0)

        start_out(j, b0, o0)

        wait_gather(a1, g1)

        @pl.when(j >= 2)
        def _():
            wait_out(b1, o1)

        transpose(a1, b1)

        @pl.when(j + 3 < SEQ)
        def _():
            fire_gather(j + 3, a1, g1)

        start_out(j + 1, b1, o1)

    wait_out(b0, o0)
    wait_out(b1, o1)


def kernel(x, table):
    t = _lookup(x.reshape(-1), table)
    a5 = t.reshape(SEQ, 8, NW, 8, 128)
    a5t = jnp.transpose(a5, (0, 1, 3, 2, 4))
    out3 = a5t.reshape(SEQ, DIM, BATCH)
    return jnp.transpose(out3, (2, 0, 1))
